# baseline (speedup 1.0000x reference)
import functools

import jax
import jax.numpy as jnp
from jax import lax
from jax.experimental import pallas as pl
from jax.experimental.pallas import tpu as pltpu
from jax.experimental.pallas import tpu_sc as plsc

_B, _S, _D, _E = 4, 8192, 768, 64
_NW = 32
_RPW = (_B * _S) // _NW
_RCHUNK = 64
_NITER = _RPW // _RCHUNK
_NV = _D // 16

_mesh = plsc.VectorSubcoreMesh(
    core_axis_name="c", subcore_axis_name="s", num_cores=2, num_subcores=16
)


@functools.partial(
    pl.kernel,
    out_type=jax.ShapeDtypeStruct((_NW, _D), jnp.float32),
    mesh=_mesh,
    scratch_types=[
        pltpu.VMEM((2, _RCHUNK * _D), jnp.float32),
        pltpu.VMEM((_D,), jnp.float32),
        pltpu.SemaphoreType.DMA,
        pltpu.SemaphoreType.DMA,
    ],
)
def _sc_mean(x_hbm, out_hbm, buf, accv, sem0, sem1):
    wid = lax.axis_index("s") * 2 + lax.axis_index("c")
    base = wid * (_RPW * _D)
    sems = (sem0, sem1)

    def dma(i, k):
        return pltpu.make_async_copy(
            x_hbm.at[pl.ds(base + i * (_RCHUNK * _D), _RCHUNK * _D)],
            buf.at[k],
            sems[k],
        )

    dma(0, 0).start()
    acc = tuple(jnp.zeros((16,), jnp.float32) for _ in range(_NV))
    for i in range(_NITER):
        k = i % 2
        if i + 1 < _NITER:
            dma(i + 1, 1 - k).start()
        dma(i, k).wait()

        def row_body(r, carry, k=k):
            off = r * _D
            return tuple(
                carry[j] + buf[k, pl.ds(off + j * 16, 16)] for j in range(_NV)
            )

        acc = lax.fori_loop(0, _RCHUNK, row_body, acc)

    for j in range(_NV):
        accv[pl.ds(j * 16, 16)] = acc[j]
    pltpu.sync_copy(accv, out_hbm.at[wid])


def _gate_tail(m, wh, bh, wo, bo):
    h = jnp.dot(m, wh, preferred_element_type=jnp.float32) + bh
    h = h * jax.nn.sigmoid(h)
    logits = jnp.dot(h, wo, preferred_element_type=jnp.float32) + bo
    iota = lax.broadcasted_iota(jnp.int32, logits.shape, 1)
    v1 = jnp.max(logits, axis=1, keepdims=True)
    i1 = jnp.min(jnp.where(logits == v1, iota, _E), axis=1, keepdims=True)
    masked = jnp.where(iota == i1, -jnp.inf, logits)
    v2 = jnp.max(masked, axis=1, keepdims=True)
    i2 = jnp.min(jnp.where(masked == v2, iota, _E), axis=1, keepdims=True)
    e2 = jnp.exp(v2 - v1)
    denom = 1.0 + e2
    w = jnp.concatenate([1.0 / denom, e2 / denom], axis=1)
    i = jnp.concatenate([i1, i2], axis=1)
    return w, i


def _tail_body(p_ref, wh_ref, bh_ref, wo_ref, bo_ref, w_ref, i_ref):
    p = p_ref[...]
    m = jnp.sum(p.reshape(_B, _NW // _B, _D), axis=1) * (1.0 / _S)
    w, i = _gate_tail(m, wh_ref[...], bh_ref[...], wo_ref[...], bo_ref[...])
    w_ref[...] = w
    i_ref[...] = i


def kernel(x, W_hidden, b_hidden, W_out, b_out):
    partials = _sc_mean(x.reshape(-1))
    bh = b_hidden.reshape(1, _D)
    bo = b_out.reshape(1, _E)
    w, i = pl.pallas_call(
        _tail_body,
        out_shape=[
            jax.ShapeDtypeStruct((_B, 2), jnp.float32),
            jax.ShapeDtypeStruct((_B, 2), jnp.int32),
        ],
    )(partials, W_hidden, bh, W_out, bo)
    return w, i
